# Initial kernel scaffold; baseline (speedup 1.0000x reference)
import functools

import jax
import jax.numpy as jnp
from jax import lax
from jax.experimental import pallas as pl
from jax.experimental.pallas import tpu as pltpu
from jax.experimental.pallas import tpu_sc as plsc

_N, _E, _D, _R = 10000, 320000, 128, 8
_EPS = 1e-5

_CHUNK = 128
_NC, _NS = 2, 16
_NW = _NC * _NS
_NCHUNKS = -(-_E // _CHUNK)
_CPW = -(-_NCHUNKS // _NW)
_EPAD = _CPW * _NW * _CHUNK
_ACC_ROWS = 10240
_ZROWS = _ACC_ROWS // _NS
_DRAIN = _N // _NS

_NBLK = 2000
_NGRID = _N // _NBLK


def _pre_body(x_ref, w1_ref, b1_ref, cw_ref, cr_ref, h_ref, root_ref, y_ref):
    r = pl.program_id(1)

    @pl.when(r == 0)
    def _():
        h = jnp.dot(x_ref[...], w1_ref[...],
                    preferred_element_type=jnp.float32) + b1_ref[...]
        h_ref[...] = h
        root_ref[...] = jnp.dot(h, cr_ref[...],
                                preferred_element_type=jnp.float32)

    y_ref[0] = jnp.dot(h_ref[...], cw_ref[0],
                       preferred_element_type=jnp.float32)


def _tc_pre(x, W1, b1, conv_w, conv_root):
    return pl.pallas_call(
        _pre_body,
        grid=(_NGRID, _R),
        in_specs=[
            pl.BlockSpec((_NBLK, _D), lambda n, r: (n, 0)),
            pl.BlockSpec((_D, _D), lambda n, r: (0, 0)),
            pl.BlockSpec((1, _D), lambda n, r: (0, 0)),
            pl.BlockSpec((1, _D, _D), lambda n, r: (r, 0, 0)),
            pl.BlockSpec((_D, _D), lambda n, r: (0, 0)),
        ],
        out_specs=[
            pl.BlockSpec((_NBLK, _D), lambda n, r: (n, 0)),
            pl.BlockSpec((_NBLK, _D), lambda n, r: (n, 0)),
            pl.BlockSpec((1, _NBLK, _D), lambda n, r: (r, n, 0)),
        ],
        out_shape=[
            jax.ShapeDtypeStruct((_N, _D), jnp.float32),
            jax.ShapeDtypeStruct((_N, _D), jnp.float32),
            jax.ShapeDtypeStruct((_R, _N, _D), jnp.float32),
        ],
    )(x, W1, b1.reshape(1, _D), conv_w, conv_root)


def _mid_body(p_ref, root_ref, cb_ref, g_ref, b_ref, cr_ref, cw_ref,
              h_ref, root2_ref, y_ref):
    r = pl.program_id(0)

    @pl.when(r == 0)
    def _():
        a = p_ref[0] + p_ref[1] + root_ref[...] + cb_ref[...]
        mean = jnp.mean(a, axis=0, keepdims=True)
        var = jnp.mean((a - mean) ** 2, axis=0, keepdims=True)
        h = (a - mean) / jnp.sqrt(var + _EPS) * g_ref[...] + b_ref[...]
        h = jnp.maximum(h, 0.0)
        h_ref[...] = h
        root2_ref[...] = jnp.dot(h, cr_ref[...],
                                 preferred_element_type=jnp.float32)

    y_ref[0] = jnp.dot(h_ref[...], cw_ref[0],
                       preferred_element_type=jnp.float32)


def _tc_mid(p, root, conv_b, bn_gamma, bn_beta, conv_root, conv_w):
    return pl.pallas_call(
        _mid_body,
        grid=(_R,),
        in_specs=[
            pl.BlockSpec((_NC, _N, _D), lambda r: (0, 0, 0)),
            pl.BlockSpec((_N, _D), lambda r: (0, 0)),
            pl.BlockSpec((1, _D), lambda r: (0, 0)),
            pl.BlockSpec((1, _D), lambda r: (0, 0)),
            pl.BlockSpec((1, _D), lambda r: (0, 0)),
            pl.BlockSpec((_D, _D), lambda r: (0, 0)),
            pl.BlockSpec((1, _D, _D), lambda r: (r, 0, 0)),
        ],
        out_specs=[
            pl.BlockSpec((_N, _D), lambda r: (0, 0)),
            pl.BlockSpec((_N, _D), lambda r: (0, 0)),
            pl.BlockSpec((1, _N, _D), lambda r: (r, 0, 0)),
        ],
        out_shape=[
            jax.ShapeDtypeStruct((_N, _D), jnp.float32),
            jax.ShapeDtypeStruct((_N, _D), jnp.float32),
            jax.ShapeDtypeStruct((_R, _N, _D), jnp.float32),
        ],
    )(p, root, conv_b.reshape(1, _D), bn_gamma.reshape(1, _D),
      bn_beta.reshape(1, _D), conv_root, conv_w)


def _fin_body(p_ref, root_ref, cb_ref, g_ref, b_ref, w2_ref, b2_ref,
              wo_ref, bo_ref, out_ref):
    a = p_ref[0] + p_ref[1] + root_ref[...] + cb_ref[...]
    mean = jnp.mean(a, axis=0, keepdims=True)
    var = jnp.mean((a - mean) ** 2, axis=0, keepdims=True)
    h = (a - mean) / jnp.sqrt(var + _EPS) * g_ref[...] + b_ref[...]
    h = jnp.maximum(h, 0.0)
    h = jnp.dot(h, w2_ref[...], preferred_element_type=jnp.float32) + b2_ref[...]
    o = jnp.dot(h, wo_ref[...], preferred_element_type=jnp.float32) + bo_ref[...]
    out_ref[...] = jax.nn.sigmoid(o)


def _tc_fin(p, root, conv_b, bn_gamma, bn_beta, W2, b2, WO, bO):
    return pl.pallas_call(
        _fin_body,
        grid=(1,),
        in_specs=[
            pl.BlockSpec((_NC, _N, _D), lambda i: (0, 0, 0)),
            pl.BlockSpec((_N, _D), lambda i: (0, 0)),
            pl.BlockSpec((1, _D), lambda i: (0, 0)),
            pl.BlockSpec((1, _D), lambda i: (0, 0)),
            pl.BlockSpec((1, _D), lambda i: (0, 0)),
            pl.BlockSpec((_D, _D), lambda i: (0, 0)),
            pl.BlockSpec((1, _D), lambda i: (0, 0)),
            pl.BlockSpec((_D, 1), lambda i: (0, 0)),
            pl.BlockSpec((1, 1), lambda i: (0, 0)),
        ],
        out_specs=pl.BlockSpec((_N, 1), lambda i: (0, 0)),
        out_shape=jax.ShapeDtypeStruct((_N, 1), jnp.float32),
    )(p, root, conv_b.reshape(1, _D), bn_gamma.reshape(1, _D),
      bn_beta.reshape(1, _D), W2, b2.reshape(1, _D), WO, bO.reshape(1, 1))


def _sc_edge_pass(y_flat, gi3, di3, zrows):
    mesh = plsc.VectorSubcoreMesh(core_axis_name="c", subcore_axis_name="s")

    @functools.partial(
        pl.kernel,
        mesh=mesh,
        out_type=jax.ShapeDtypeStruct((_NC, _N, _D), jnp.float32),
        scratch_types=[
            pltpu.VMEM((_CPW, _CHUNK), jnp.int32),
            pltpu.VMEM((_CPW, _CHUNK), jnp.int32),
            pltpu.VMEM((_CHUNK, _D), jnp.float32),
            pltpu.VMEM_SHARED((_ACC_ROWS, _D), jnp.float32),
            pltpu.SemaphoreType.DMA,
        ],
    )
    def k(y_hbm, gi_hbm, di_hbm, z_hbm, out_hbm, gi_v, di_v, buf, acc, sem):
        c = lax.axis_index("c")
        s = lax.axis_index("s")
        wid = s * _NC + c
        pltpu.sync_copy(z_hbm, acc.at[pl.ds(s * _ZROWS, _ZROWS)])
        pltpu.sync_copy(gi_hbm.at[wid], gi_v)
        pltpu.sync_copy(di_hbm.at[wid], di_v)
        plsc.subcore_barrier()

        @pl.loop(0, _CPW)
        def _(ci):
            pltpu.sync_copy(y_hbm.at[gi_v.at[ci]], buf)
            pltpu.sync_copy(buf, acc.at[di_v.at[ci]], add=True)

        plsc.subcore_barrier()
        pltpu.sync_copy(acc.at[pl.ds(s * _DRAIN, _DRAIN)],
                        out_hbm.at[c, pl.ds(s * _DRAIN, _DRAIN)])

    return k(y_flat, gi3, di3, zrows)


def kernel(x, edge_index, edge_type, W1, b1, conv_w, conv_root, conv_b,
           bn_gamma, bn_beta, W2, b2, WO, bO):
    src = edge_index[0]
    dst = edge_index[1]
    gidx = edge_type * _N + src
    pad = _EPAD - _E
    gidx_p = jnp.concatenate([gidx, jnp.zeros((pad,), jnp.int32)])
    dummy = _N + (jnp.arange(pad, dtype=jnp.int32) % (_ACC_ROWS - _N))
    dst_p = jnp.concatenate([dst, dummy])
    gi3 = gidx_p.reshape(_NW, _CPW, _CHUNK)
    di3 = dst_p.reshape(_NW, _CPW, _CHUNK)
    zrows = jnp.zeros((_ZROWS, _D), jnp.float32)

    h1, root1, y1 = _tc_pre(x, W1, b1, conv_w, conv_root)
    p1 = _sc_edge_pass(y1.reshape(_R * _N, _D), gi3, di3, zrows)
    h2, root2, y2 = _tc_mid(p1, root1, conv_b, bn_gamma, bn_beta,
                            conv_root, conv_w)
    p2 = _sc_edge_pass(y2.reshape(_R * _N, _D), gi3, di3, zrows)
    return _tc_fin(p2, root2, conv_b, bn_gamma, bn_beta, W2, b2, WO, bO)


# speedup vs baseline: 18.3166x; 18.3166x over previous
import functools

import jax
import jax.numpy as jnp
from jax import lax
from jax.experimental import pallas as pl
from jax.experimental.pallas import tpu as pltpu
from jax.experimental.pallas import tpu_sc as plsc

_N, _E, _D, _R = 10000, 320000, 128, 8
_EPS = 1e-5

_CHUNK = 128
_NC, _NS = 2, 16
_NW = _NC * _NS
_NCHUNKS = -(-_E // _CHUNK)
_CPW = -(-_NCHUNKS // _NW)
_EPAD = _CPW * _NW * _CHUNK
_ACC_ROWS = 10240
_ZROWS = _ACC_ROWS // _NS

_NBLK = 2000
_NGRID = _N // _NBLK


def _pre_body(x_ref, w1_ref, b1_ref, cw_ref, cr_ref, h_ref, root_ref, y_ref):
    r = pl.program_id(1)

    @pl.when(r == 0)
    def _():
        h = jnp.dot(x_ref[...], w1_ref[...],
                    preferred_element_type=jnp.float32) + b1_ref[...]
        h_ref[...] = h
        root_ref[...] = jnp.dot(h, cr_ref[...],
                                preferred_element_type=jnp.float32)

    y_ref[0] = jnp.dot(h_ref[...], cw_ref[0],
                       preferred_element_type=jnp.float32)


def _tc_pre(x, W1, b1, conv_w, conv_root):
    return pl.pallas_call(
        _pre_body,
        grid=(_NGRID, _R),
        in_specs=[
            pl.BlockSpec((_NBLK, _D), lambda n, r: (n, 0)),
            pl.BlockSpec((_D, _D), lambda n, r: (0, 0)),
            pl.BlockSpec((1, _D), lambda n, r: (0, 0)),
            pl.BlockSpec((1, _D, _D), lambda n, r: (r, 0, 0)),
            pl.BlockSpec((_D, _D), lambda n, r: (0, 0)),
        ],
        out_specs=[
            pl.BlockSpec((_NBLK, _D), lambda n, r: (n, 0)),
            pl.BlockSpec((_NBLK, _D), lambda n, r: (n, 0)),
            pl.BlockSpec((1, _NBLK, _D), lambda n, r: (r, n, 0)),
        ],
        out_shape=[
            jax.ShapeDtypeStruct((_N, _D), jnp.float32),
            jax.ShapeDtypeStruct((_N, _D), jnp.float32),
            jax.ShapeDtypeStruct((_R, _N, _D), jnp.float32),
        ],
    )(x, W1, b1.reshape(1, _D), conv_w, conv_root)


def _mid_body(p_ref, root_ref, cb_ref, g_ref, b_ref, cr_ref, cw_ref,
              h_ref, root2_ref, y_ref):
    r = pl.program_id(0)

    @pl.when(r == 0)
    def _():
        a = p_ref[0] + p_ref[1] + root_ref[...] + cb_ref[...]
        mean = jnp.mean(a, axis=0, keepdims=True)
        var = jnp.mean((a - mean) ** 2, axis=0, keepdims=True)
        h = (a - mean) / jnp.sqrt(var + _EPS) * g_ref[...] + b_ref[...]
        h = jnp.maximum(h, 0.0)
        h_ref[...] = h
        root2_ref[...] = jnp.dot(h, cr_ref[...],
                                 preferred_element_type=jnp.float32)

    y_ref[0] = jnp.dot(h_ref[...], cw_ref[0],
                       preferred_element_type=jnp.float32)


def _tc_mid(p, root, conv_b, bn_gamma, bn_beta, conv_root, conv_w):
    return pl.pallas_call(
        _mid_body,
        grid=(_R,),
        in_specs=[
            pl.BlockSpec((_NC, _N, _D), lambda r: (0, 0, 0)),
            pl.BlockSpec((_N, _D), lambda r: (0, 0)),
            pl.BlockSpec((1, _D), lambda r: (0, 0)),
            pl.BlockSpec((1, _D), lambda r: (0, 0)),
            pl.BlockSpec((1, _D), lambda r: (0, 0)),
            pl.BlockSpec((_D, _D), lambda r: (0, 0)),
            pl.BlockSpec((1, _D, _D), lambda r: (r, 0, 0)),
        ],
        out_specs=[
            pl.BlockSpec((_N, _D), lambda r: (0, 0)),
            pl.BlockSpec((_N, _D), lambda r: (0, 0)),
            pl.BlockSpec((1, _N, _D), lambda r: (r, 0, 0)),
        ],
        out_shape=[
            jax.ShapeDtypeStruct((_N, _D), jnp.float32),
            jax.ShapeDtypeStruct((_N, _D), jnp.float32),
            jax.ShapeDtypeStruct((_R, _N, _D), jnp.float32),
        ],
    )(p, root, conv_b.reshape(1, _D), bn_gamma.reshape(1, _D),
      bn_beta.reshape(1, _D), conv_root, conv_w)


def _fin_body(p_ref, root_ref, cb_ref, g_ref, b_ref, w2_ref, b2_ref,
              wo_ref, bo_ref, out_ref):
    a = p_ref[0] + p_ref[1] + root_ref[...] + cb_ref[...]
    mean = jnp.mean(a, axis=0, keepdims=True)
    var = jnp.mean((a - mean) ** 2, axis=0, keepdims=True)
    h = (a - mean) / jnp.sqrt(var + _EPS) * g_ref[...] + b_ref[...]
    h = jnp.maximum(h, 0.0)
    h = jnp.dot(h, w2_ref[...], preferred_element_type=jnp.float32) + b2_ref[...]
    o = jnp.dot(h, wo_ref[...], preferred_element_type=jnp.float32) + bo_ref[...]
    out_ref[...] = jax.nn.sigmoid(o)


def _tc_fin(p, root, conv_b, bn_gamma, bn_beta, W2, b2, WO, bO):
    return pl.pallas_call(
        _fin_body,
        grid=(1,),
        in_specs=[
            pl.BlockSpec((_NC, _N, _D), lambda i: (0, 0, 0)),
            pl.BlockSpec((_N, _D), lambda i: (0, 0)),
            pl.BlockSpec((1, _D), lambda i: (0, 0)),
            pl.BlockSpec((1, _D), lambda i: (0, 0)),
            pl.BlockSpec((1, _D), lambda i: (0, 0)),
            pl.BlockSpec((_D, _D), lambda i: (0, 0)),
            pl.BlockSpec((1, _D), lambda i: (0, 0)),
            pl.BlockSpec((_D, 1), lambda i: (0, 0)),
            pl.BlockSpec((1, 1), lambda i: (0, 0)),
        ],
        out_specs=pl.BlockSpec((_N, 1), lambda i: (0, 0)),
        out_shape=jax.ShapeDtypeStruct((_N, 1), jnp.float32),
    )(p, root, conv_b.reshape(1, _D), bn_gamma.reshape(1, _D),
      bn_beta.reshape(1, _D), W2, b2.reshape(1, _D), WO, bO.reshape(1, 1))


def _sc_edge_pass(y_flat, gi3, di3, zrows):
    mesh = plsc.VectorSubcoreMesh(core_axis_name="c", subcore_axis_name="s")

    @functools.partial(
        pl.kernel,
        mesh=mesh,
        out_type=jax.ShapeDtypeStruct((_NC, _ACC_ROWS, _D), jnp.float32),
        scratch_types=[
            pltpu.VMEM((_CPW, _CHUNK), jnp.int32),
            pltpu.VMEM((_CPW, _CHUNK), jnp.int32),
            pltpu.VMEM((_CHUNK, _D), jnp.float32),
            pltpu.VMEM_SHARED((_ACC_ROWS, _D), jnp.float32),
            pltpu.SemaphoreType.DMA,
        ],
    )
    def k(y_hbm, gi_hbm, di_hbm, z_hbm, out_hbm, gi_v, di_v, buf, acc, sem):
        c = lax.axis_index("c")
        s = lax.axis_index("s")
        wid = s * _NC + c
        pltpu.sync_copy(z_hbm, acc.at[pl.ds(s * _ZROWS, _ZROWS)])
        pltpu.sync_copy(gi_hbm.at[wid], gi_v)
        pltpu.sync_copy(di_hbm.at[wid], di_v)
        plsc.subcore_barrier()

        @pl.loop(0, _CPW)
        def _(ci):
            pltpu.sync_copy(y_hbm.at[gi_v.at[ci]], buf)
            pltpu.sync_copy(buf, acc.at[di_v.at[ci]], add=True)

        plsc.subcore_barrier()
        pltpu.sync_copy(acc.at[pl.ds(s * _ZROWS, _ZROWS)],
                        out_hbm.at[c, pl.ds(s * _ZROWS, _ZROWS)])

    return k(y_flat, gi3, di3, zrows)


def kernel(x, edge_index, edge_type, W1, b1, conv_w, conv_root, conv_b,
           bn_gamma, bn_beta, W2, b2, WO, bO):
    src = edge_index[0]
    dst = edge_index[1]
    gidx = edge_type * _N + src
    pad = _EPAD - _E
    gidx_p = jnp.concatenate([gidx, jnp.zeros((pad,), jnp.int32)])
    dummy = _N + (jnp.arange(pad, dtype=jnp.int32) % (_ACC_ROWS - _N))
    dst_p = jnp.concatenate([dst, dummy])
    gi3 = gidx_p.reshape(_NW, _CPW, _CHUNK)
    di3 = dst_p.reshape(_NW, _CPW, _CHUNK)
    zrows = jnp.zeros((_ZROWS, _D), jnp.float32)

    h1, root1, y1 = _tc_pre(x, W1, b1, conv_w, conv_root)
    p1 = _sc_edge_pass(y1.reshape(_R * _N, _D), gi3, di3, zrows)
    h2, root2, y2 = _tc_mid(p1, root1, conv_b, bn_gamma, bn_beta,
                            conv_root, conv_w)
    p2 = _sc_edge_pass(y2.reshape(_R * _N, _D), gi3, di3, zrows)
    return _tc_fin(p2, root2, conv_b, bn_gamma, bn_beta, W2, b2, WO, bO)
